# baseline (speedup 1.0000x reference)
import functools

import jax
import jax.numpy as jnp
import numpy as np
from jax import lax
from jax.experimental import pallas as pl
from jax.experimental.pallas import tpu as pltpu
from jax.experimental.pallas import tpu_sc as plsc

NV = 40962
NF = 81920
B = 4
IN_CH = 32
NECK = 16
OUT_CH = 32
EPS = 1e-5

NVP = 43008
TR = 1024
CH = 8
OT = 64
NC = 2
NS = 16
NW = NC * NS
NBUF = 4

f32 = jnp.float32
i32 = jnp.int32

_PAD_L = np.asarray((np.arange((NVP - NV) * 7) * 193) % NV, np.int32)
_PAD_F = np.asarray((np.arange((NVP - NV) * 6) * 193) % NF, np.int32)
_ZW_L = np.zeros((NVP - NV) * 7, np.float32)
_ZW_F = np.zeros((NVP - NV) * 6, np.float32)


def _make_sc_gather(t_rows, din, r_rows, k_nnz, n_w, nbuf=NBUF):
    dout = din * n_w
    rows_pw = r_rows // NW
    chunks_pw = rows_pw // CH
    chunks_pt = OT // CH
    mesh = plsc.VectorSubcoreMesh(core_axis_name="c", subcore_axis_name="s")
    cw = CH * k_nnz
    rowlen = cw * n_w

    def body(table, idxh, *rest):
        whs = rest[:n_w]
        outh = rest[n_w]
        idx_v = rest[n_w + 1]
        w_vs = rest[n_w + 2:2 * n_w + 2]
        rows_v = rest[2 * n_w + 2]
        out_v = rest[2 * n_w + 3]
        sems = rest[2 * n_w + 4:]
        cid = lax.axis_index("c")
        sid = lax.axis_index("s")
        wid = sid * NC + cid
        pltpu.sync_copy(idxh.at[pl.ds(wid * rows_pw * k_nnz,
                                      rows_pw * k_nnz)], idx_v)
        for s_ in range(n_w):
            pltpu.sync_copy(whs[s_].at[pl.ds(wid * rows_pw * k_nnz,
                                             rows_pw * k_nnz)], w_vs[s_])

        def start(cl, b):
            pltpu.async_copy(table.at[idx_v.at[pl.ds(cl * cw, cw)]],
                             rows_v.at[b], sems[b])

        def wait(cl, b):
            pltpu.make_async_copy(table.at[idx_v.at[pl.ds(cl * cw, cw)]],
                                  rows_v.at[b], sems[b]).wait()

        offs = list(range(0, max(cw - 15, 1), 16))
        if cw % 16:
            offs.append(cw - 16)
        nv = din // 16

        for b in range(nbuf):
            start(b, b)

        def pair_body(p, _):
            for b in range(nbuf):
                cl = p * nbuf + b
                wait(cl, b)
                wbase = cl * cw
                wvecs = [[w_vs[s_][pl.ds(wbase + o, 16)] for o in offs]
                         for s_ in range(n_w)]

                def wscal(s_, j):
                    if j >= offs[-1]:
                        return wvecs[s_][-1][j - offs[-1]]
                    return wvecs[s_][j // 16][j % 16]

                orow = lax.rem(cl, chunks_pt) * CH
                for r in range(CH):
                    accs = [[None] * nv for _ in range(n_w)]
                    for kk in range(k_nnz):
                        ws = [wscal(s_, r * k_nnz + kk) for s_ in range(n_w)]
                        for v in range(nv):
                            rv = rows_v[b, r * k_nnz + kk, pl.ds(v * 16, 16)]
                            for w in range(n_w):
                                pr = ws[w] * rv
                                accs[w][v] = pr if kk == 0 else accs[w][v] + pr
                    for w in range(n_w):
                        for v in range(nv):
                            out_v[orow + r,
                                  pl.ds(w * din + v * 16, 16)] = accs[w][v]

                @pl.when(cl + nbuf < chunks_pw)
                def _():
                    start(cl + nbuf, b)

                @pl.when(lax.rem(cl, chunks_pt) == chunks_pt - 1)
                def _():
                    t = lax.div(cl, chunks_pt)
                    pltpu.sync_copy(
                        out_v, outh.at[pl.ds(wid * rows_pw + t * OT, OT)])
            return 0

        lax.fori_loop(0, chunks_pw // nbuf, pair_body, 0)

    return functools.partial(
        pl.kernel,
        out_type=jax.ShapeDtypeStruct((r_rows, dout), f32),
        mesh=mesh,
        scratch_types=[
            pltpu.VMEM((rows_pw * k_nnz,), i32),
        ] + [pltpu.VMEM((rows_pw * k_nnz,), f32)] * n_w + [
            pltpu.VMEM((nbuf, cw, din), f32),
            pltpu.VMEM((OT, dout), f32),
        ] + [pltpu.SemaphoreType.DMA] * nbuf,
        compiler_params=pltpu.CompilerParams(use_tc_tiling_on_sc=False),
    )(body)


def _make_sc_gather_g(nbuf=NBUF):
    rows_pw = NF // NW
    chunks_pw = rows_pw // CH
    chunks_pt = OT // CH
    seg = rows_pw * 3
    mesh = plsc.VectorSubcoreMesh(core_axis_name="c", subcore_axis_name="s")

    def body(table, colsh, valsh, ewh, nsh, outh,
             idx_v, idxc_v, gw_v, ew_v, ns_v, rows_v, out_v, *sems):
        cid = lax.axis_index("c")
        sid = lax.axis_index("s")
        wid = sid * NC + cid
        for d in range(3):
            pltpu.sync_copy(colsh.at[pl.ds(d * 3 * NF + wid * seg, seg)],
                            idx_v.at[d])
            pltpu.sync_copy(valsh.at[pl.ds(d * 3 * NF + wid * seg, seg)],
                            gw_v.at[d])
        pltpu.sync_copy(ewh.at[pl.ds(wid * seg, seg)], ew_v)
        pltpu.sync_copy(nsh.at[pl.ds(wid * seg, seg)], ns_v)

        def il_body(cl, _):
            for d in range(3):
                a = idx_v[d, pl.ds(cl * 24, 16)]
                bq = idx_v[d, pl.ds(cl * 24 + 8, 16)]
                idxc_v[pl.ds(cl * 72 + d * 24, 16)] = a
                idxc_v[pl.ds(cl * 72 + d * 24 + 8, 16)] = bq
            return 0

        lax.fori_loop(0, chunks_pw, il_body, 0)

        def start(cl, b):
            pltpu.async_copy(table.at[idxc_v.at[pl.ds(cl * 72, 72)]],
                             rows_v.at[b], sems[b])

        def wait(cl, b):
            pltpu.make_async_copy(table.at[idxc_v.at[pl.ds(cl * 72, 72)]],
                                  rows_v.at[b], sems[b]).wait()

        for b in range(nbuf):
            start(b, b)

        def sc24(pair, j):
            return pair[0][j] if j < 16 else pair[1][j - 8]

        def pair_body(p, _):
            for b in range(nbuf):
                cl = p * nbuf + b
                wait(cl, b)
                gw = [(gw_v[d, pl.ds(cl * 24, 16)],
                       gw_v[d, pl.ds(cl * 24 + 8, 16)]) for d in range(3)]
                ew = (ew_v[pl.ds(cl * 24, 16)], ew_v[pl.ds(cl * 24 + 8, 16)])
                ns = (ns_v[pl.ds(cl * 24, 16)], ns_v[pl.ds(cl * 24 + 8, 16)])

                orow = lax.rem(cl, chunks_pt) * CH
                for r in range(CH):
                    acc_e = [None] * 4
                    acc_n = [None] * 4
                    for d in range(3):
                        ewd = sc24(ew, r * 3 + d)
                        nsd = sc24(ns, r * 3 + d)
                        for j in range(3):
                            gvw = sc24(gw[d], r * 3 + j)
                            we = gvw * ewd
                            wn = gvw * nsd
                            for v in range(4):
                                rv = rows_v[b, d * 24 + r * 3 + j,
                                            pl.ds(v * 16, 16)]
                                pe = we * rv
                                pn = wn * rv
                                if acc_e[v] is None:
                                    acc_e[v] = pe
                                    acc_n[v] = pn
                                else:
                                    acc_e[v] = acc_e[v] + pe
                                    acc_n[v] = acc_n[v] + pn
                    for v in range(4):
                        out_v[orow + r, pl.ds(v * 16, 16)] = acc_e[v]
                        out_v[orow + r, pl.ds(64 + v * 16, 16)] = acc_n[v]

                @pl.when(cl + nbuf < chunks_pw)
                def _():
                    start(cl + nbuf, b)

                @pl.when(lax.rem(cl, chunks_pt) == chunks_pt - 1)
                def _():
                    t = lax.div(cl, chunks_pt)
                    pltpu.sync_copy(
                        out_v, outh.at[pl.ds(wid * rows_pw + t * OT, OT)])
            return 0

        lax.fori_loop(0, chunks_pw // nbuf, pair_body, 0)

    return functools.partial(
        pl.kernel,
        out_type=jax.ShapeDtypeStruct((NF, 128), f32),
        mesh=mesh,
        scratch_types=[
            pltpu.VMEM((3, seg), i32),
            pltpu.VMEM((3 * seg,), i32),
            pltpu.VMEM((3, seg), f32),
            pltpu.VMEM((seg,), f32),
            pltpu.VMEM((seg,), f32),
            pltpu.VMEM((nbuf, 72, 64), f32),
            pltpu.VMEM((OT, 128), f32),
        ] + [pltpu.SemaphoreType.DMA] * nbuf,
        compiler_params=pltpu.CompilerParams(use_tc_tiling_on_sc=False),
    )(body)


def _k1_body(x_ref, g_ref, s_ref):
    i = pl.program_id(0)
    xt = jnp.transpose(x_ref[...].reshape(128, TR))
    rows = lax.broadcasted_iota(i32, (TR, 128), 0) + i * TR
    xt = jnp.where(rows < NV, xt, 0.0)
    g = lax.dot_general(xt, xt, (((0,), (0,)), ((), ())),
                        preferred_element_type=f32)
    s = jnp.sum(xt, axis=0, keepdims=True)
    spad = jnp.concatenate([s, jnp.zeros((7, 128), f32)], axis=0)

    @pl.when(i == 0)
    def _():
        g_ref[...] = g
        s_ref[...] = spad

    @pl.when(i > 0)
    def _():
        g_ref[...] += g
        s_ref[...] += spad


def _k2_body(x_ref, w_ref, b_ref, h_ref):
    i = pl.program_id(0)
    xt = jnp.transpose(x_ref[...].reshape(128, TR))
    rows = lax.broadcasted_iota(i32, (TR, 64), 0) + i * TR
    h = jnp.dot(xt, w_ref[...], preferred_element_type=f32) + b_ref[0:1, :]
    h_ref[...] = jnp.where(rows < NV, jnp.maximum(h, 0.0), 0.0)


def _k6_body(h2_ref, lap_ref, gv_ref, kid_ref, klap_ref, kew_ref, kns_ref,
             y_ref, st_ref):
    i = pl.program_id(0)
    gv = gv_ref[...]
    y = (jnp.dot(h2_ref[...], kid_ref[...], preferred_element_type=f32)
         + jnp.dot(lap_ref[...], klap_ref[...], preferred_element_type=f32)
         + jnp.dot(gv[:, :64], kew_ref[...], preferred_element_type=f32)
         + jnp.dot(gv[:, 64:], kns_ref[...], preferred_element_type=f32))
    y_ref[...] = y
    st = jnp.concatenate([jnp.sum(y, axis=0, keepdims=True),
                          jnp.sum(y * y, axis=0, keepdims=True),
                          jnp.zeros((6, 64), f32)], axis=0)

    @pl.when(i == 0)
    def _():
        st_ref[...] = st

    @pl.when(i > 0)
    def _():
        st_ref[...] += st


def _k7_body(y_ref, s2_ref, t2_ref, w3_ref, b3_ref, z_ref, st_ref):
    i = pl.program_id(0)
    h3 = jnp.maximum(y_ref[...] * s2_ref[0:1, :] + t2_ref[0:1, :], 0.0)
    z = jnp.dot(h3, w3_ref[...], preferred_element_type=f32) + b3_ref[0:1, :]
    rows = lax.broadcasted_iota(i32, (TR, 128), 0) + i * TR
    z = jnp.where(rows < NV, z, 0.0)
    z_ref[...] = z
    st = jnp.concatenate([jnp.sum(z, axis=0, keepdims=True),
                          jnp.sum(z * z, axis=0, keepdims=True),
                          jnp.zeros((6, 128), f32)], axis=0)

    @pl.when(i == 0)
    def _():
        st_ref[...] = st

    @pl.when(i > 0)
    def _():
        st_ref[...] += st


def _k8_body(z_ref, x_ref, s3_ref, t3_ref, o_ref):
    xt = jnp.transpose(x_ref[...].reshape(128, TR))
    o = jnp.maximum(z_ref[...] * s3_ref[0:1, :] + t3_ref[0:1, :] + xt, 0.0)
    o_ref[...] = jnp.transpose(o).reshape(4, 32, TR)


def _row_spec(w):
    return pl.BlockSpec((TR, w), lambda i: (i, 0))


def _full_spec(h, w):
    return pl.BlockSpec((h, w), lambda i: (0, 0))


_GRID = NVP // TR
_GRIDX = -(-NV // TR)


def _tc_call(body, in_specs, out_specs, out_shapes, grid=_GRID):
    return pl.pallas_call(
        body,
        grid=(grid,),
        in_specs=in_specs,
        out_specs=out_specs,
        out_shape=out_shapes,
        compiler_params=pltpu.CompilerParams(
            dimension_semantics=("arbitrary",)),
    )


def _x_spec(clamp=None):
    if clamp is None:
        return pl.BlockSpec((B, IN_CH, TR), lambda i: (0, 0, i))
    return pl.BlockSpec((B, IN_CH, TR),
                        lambda i: (0, 0, jnp.minimum(i, clamp)))


def kernel(x, W1a, b1a, coeffs, W3a, b3a, g1a, be1a, g2a, be2a, g3a, be3a,
           G_rows, G_cols, G_vals, L_rows, L_cols, L_vals,
           F_rows, F_cols, F_vals, EW, NS_):
    N = B * NV
    eyeB = jnp.eye(B, dtype=f32)

    g128, csum8 = _tc_call(
        _k1_body,
        [_x_spec()],
        [_full_spec(128, 128), _full_spec(8, 128)],
        [jax.ShapeDtypeStruct((128, 128), f32),
         jax.ShapeDtypeStruct((8, 128), f32)],
        grid=_GRIDX,
    )(x)
    csum = csum8[0]

    mu_x = csum.reshape(B, IN_CH).sum(0) / N
    Sig = sum(g128[b * IN_CH:(b + 1) * IN_CH, b * IN_CH:(b + 1) * IN_CH]
              for b in range(B)) / N
    mu_h = W1a @ mu_x + b1a
    Eh2 = jnp.einsum('ci,ij,cj->c', W1a, Sig, W1a) + 2 * b1a * (W1a @ mu_x) + b1a ** 2
    s1 = g1a / jnp.sqrt(Eh2 - mu_h ** 2 + EPS)
    W1K = jnp.kron(eyeB, (W1a * s1[:, None]).T)
    b1K = jnp.tile(s1 * (b1a - mu_h) + be1a, B)
    b1K8 = jnp.tile(b1K[None, :], (8, 1))

    (h2,) = _tc_call(
        _k2_body,
        [_x_spec(clamp=_GRIDX - 1), _full_spec(128, 64), _full_spec(8, 64)],
        [_row_spec(64)],
        [jax.ShapeDtypeStruct((NVP, 64), f32)],
    )(x, W1K, b1K8)

    ew_flat = EW.reshape(-1)
    ns_flat = NS_.reshape(-1)

    idxL = jnp.concatenate([L_cols, jnp.asarray(_PAD_L)])
    wL = jnp.concatenate([L_vals, jnp.asarray(_ZW_L)])
    idxF = jnp.concatenate([F_cols, jnp.asarray(_PAD_F)])
    wF = jnp.concatenate([F_vals, jnp.asarray(_ZW_F)])

    lap = _make_sc_gather(NVP, 64, NVP, 7, 1)(h2, idxL, wL)
    gf = _make_sc_gather_g()(h2, G_cols, G_vals, ew_flat, ns_flat)
    gv = _make_sc_gather(NF, 128, NVP, 6, 1)(gf, idxF, wF)

    Ks = [jnp.kron(eyeB, coeffs[j::4, :]) for j in range(4)]
    y, st6 = _tc_call(
        _k6_body,
        [_row_spec(64), _row_spec(64), _row_spec(128),
         _full_spec(64, 64), _full_spec(64, 64), _full_spec(64, 64),
         _full_spec(64, 64)],
        [_row_spec(64), _full_spec(8, 64)],
        [jax.ShapeDtypeStruct((NVP, 64), f32),
         jax.ShapeDtypeStruct((8, 64), f32)],
    )(h2, lap, gv, Ks[0], Ks[1], Ks[2], Ks[3])

    mu_y = st6[0].reshape(B, NECK).sum(0) / N
    var_y = st6[1].reshape(B, NECK).sum(0) / N - mu_y ** 2
    s2 = g2a / jnp.sqrt(var_y + EPS)
    t2 = -mu_y * s2 + be2a
    s2c8 = jnp.tile(jnp.tile(s2, B)[None, :], (8, 1))
    t2c8 = jnp.tile(jnp.tile(t2, B)[None, :], (8, 1))

    W3K = jnp.kron(eyeB, W3a.T)
    b3K8 = jnp.tile(jnp.tile(b3a, B)[None, :], (8, 1))
    z, st7 = _tc_call(
        _k7_body,
        [_row_spec(64), _full_spec(8, 64), _full_spec(8, 64),
         _full_spec(64, 128), _full_spec(8, 128)],
        [_row_spec(128), _full_spec(8, 128)],
        [jax.ShapeDtypeStruct((NVP, 128), f32),
         jax.ShapeDtypeStruct((8, 128), f32)],
    )(y, s2c8, t2c8, W3K, b3K8)

    mu_z = st7[0].reshape(B, OUT_CH).sum(0) / N
    var_z = st7[1].reshape(B, OUT_CH).sum(0) / N - mu_z ** 2
    s3 = g3a / jnp.sqrt(var_z + EPS)
    t3 = -mu_z * s3 + be3a
    s3c8 = jnp.tile(jnp.tile(s3, B)[None, :], (8, 1))
    t3c8 = jnp.tile(jnp.tile(t3, B)[None, :], (8, 1))

    (out,) = _tc_call(
        _k8_body,
        [_row_spec(128), _x_spec(), _full_spec(8, 128), _full_spec(8, 128)],
        [pl.BlockSpec((B, OUT_CH, TR), lambda i: (0, 0, i))],
        [jax.ShapeDtypeStruct((B, OUT_CH, NV), f32)],
        grid=_GRIDX,
    )(z, x, s3c8, t3c8)

    return out


# speedup vs baseline: 1.0193x; 1.0193x over previous
import functools

import jax
import jax.numpy as jnp
import numpy as np
from jax import lax
from jax.experimental import pallas as pl
from jax.experimental.pallas import tpu as pltpu
from jax.experimental.pallas import tpu_sc as plsc

NV = 40962
NF = 81920
B = 4
IN_CH = 32
NECK = 16
OUT_CH = 32
EPS = 1e-5

NVP = 43008
TR = 1024
CH = 8
OT = 64
NC = 2
NS = 16
NW = NC * NS
NBUF = 4

f32 = jnp.float32
i32 = jnp.int32

_PAD_L = np.asarray((np.arange((NVP - NV) * 7) * 193) % NV, np.int32)
_PAD_F = np.asarray((np.arange((NVP - NV) * 6) * 193) % NF, np.int32)
_ZW_L = np.zeros((NVP - NV) * 7, np.float32)
_ZW_F = np.zeros((NVP - NV) * 6, np.float32)


def _make_sc_gather(t_rows, din, r_rows, k_nnz, n_w, nbuf=NBUF):
    dout = din * n_w
    rows_pw = r_rows // NW
    chunks_pw = rows_pw // CH
    chunks_pt = OT // CH
    mesh = plsc.VectorSubcoreMesh(core_axis_name="c", subcore_axis_name="s")
    cw = CH * k_nnz
    rowlen = cw * n_w

    def body(table, idxh, *rest):
        whs = rest[:n_w]
        outh = rest[n_w]
        idx_v = rest[n_w + 1]
        w_vs = rest[n_w + 2:2 * n_w + 2]
        rows_v = rest[2 * n_w + 2]
        out_v = rest[2 * n_w + 3]
        sems = rest[2 * n_w + 4:]
        cid = lax.axis_index("c")
        sid = lax.axis_index("s")
        wid = sid * NC + cid
        pltpu.sync_copy(idxh.at[pl.ds(wid * rows_pw * k_nnz,
                                      rows_pw * k_nnz)], idx_v)
        for s_ in range(n_w):
            pltpu.sync_copy(whs[s_].at[pl.ds(wid * rows_pw * k_nnz,
                                             rows_pw * k_nnz)], w_vs[s_])

        def start(cl, b):
            pltpu.async_copy(table.at[idx_v.at[pl.ds(cl * cw, cw)]],
                             rows_v.at[b], sems[b])

        def wait(cl, b):
            pltpu.make_async_copy(table.at[idx_v.at[pl.ds(cl * cw, cw)]],
                                  rows_v.at[b], sems[b]).wait()

        offs = list(range(0, max(cw - 15, 1), 16))
        if cw % 16:
            offs.append(cw - 16)
        nv = din // 16

        for b in range(nbuf):
            start(b, b)

        def pair_body(p, _):
            for b in range(nbuf):
                cl = p * nbuf + b
                wait(cl, b)
                wbase = cl * cw
                wvecs = [[w_vs[s_][pl.ds(wbase + o, 16)] for o in offs]
                         for s_ in range(n_w)]

                def wscal(s_, j):
                    if j >= offs[-1]:
                        return wvecs[s_][-1][j - offs[-1]]
                    return wvecs[s_][j // 16][j % 16]

                orow = lax.rem(cl, chunks_pt) * CH
                for r in range(CH):
                    accs = [[None] * nv for _ in range(n_w)]
                    for kk in range(k_nnz):
                        ws = [wscal(s_, r * k_nnz + kk) for s_ in range(n_w)]
                        for v in range(nv):
                            rv = rows_v[b, r * k_nnz + kk, pl.ds(v * 16, 16)]
                            for w in range(n_w):
                                pr = ws[w] * rv
                                accs[w][v] = pr if kk == 0 else accs[w][v] + pr
                    for w in range(n_w):
                        for v in range(nv):
                            out_v[orow + r,
                                  pl.ds(w * din + v * 16, 16)] = accs[w][v]

                @pl.when(cl + nbuf < chunks_pw)
                def _():
                    start(cl + nbuf, b)

                @pl.when(lax.rem(cl, chunks_pt) == chunks_pt - 1)
                def _():
                    t = lax.div(cl, chunks_pt)
                    pltpu.sync_copy(
                        out_v, outh.at[pl.ds(wid * rows_pw + t * OT, OT)])
            return 0

        lax.fori_loop(0, chunks_pw // nbuf, pair_body, 0)

    return functools.partial(
        pl.kernel,
        out_type=jax.ShapeDtypeStruct((r_rows, dout), f32),
        mesh=mesh,
        scratch_types=[
            pltpu.VMEM((rows_pw * k_nnz,), i32),
        ] + [pltpu.VMEM((rows_pw * k_nnz,), f32)] * n_w + [
            pltpu.VMEM((nbuf, cw, din), f32),
            pltpu.VMEM((OT, dout), f32),
        ] + [pltpu.SemaphoreType.DMA] * nbuf,
        compiler_params=pltpu.CompilerParams(use_tc_tiling_on_sc=False),
    )(body)


def _make_sc_gather_g(nbuf=NBUF):
    rows_pw = NF // NW
    chunks_pw = rows_pw // CH
    chunks_pt = OT // CH
    seg = rows_pw * 3
    mesh = plsc.VectorSubcoreMesh(core_axis_name="c", subcore_axis_name="s")

    def body(table, colsh, valsh, ewh, nsh, outh,
             idx_v, gw_v, ew_v, ns_v, rows_v, out_v, *sems):
        cid = lax.axis_index("c")
        sid = lax.axis_index("s")
        wid = sid * NC + cid
        for d in range(3):
            pltpu.sync_copy(colsh.at[pl.ds(d * 3 * NF + wid * seg, seg)],
                            idx_v.at[d])
            pltpu.sync_copy(valsh.at[pl.ds(d * 3 * NF + wid * seg, seg)],
                            gw_v.at[d])
        pltpu.sync_copy(ewh.at[pl.ds(wid * seg, seg)], ew_v)
        pltpu.sync_copy(nsh.at[pl.ds(wid * seg, seg)], ns_v)

        def start(cl, b):
            for d in range(3):
                pltpu.async_copy(
                    table.at[idx_v.at[d].at[pl.ds(cl * 24, 24)]],
                    rows_v.at[b].at[pl.ds(d * 24, 24)], sems[b])

        def wait(cl, b):
            for d in range(3):
                pltpu.make_async_copy(
                    table.at[idx_v.at[d].at[pl.ds(cl * 24, 24)]],
                    rows_v.at[b].at[pl.ds(d * 24, 24)], sems[b]).wait()

        for b in range(nbuf):
            start(b, b)

        def sc24(pair, j):
            return pair[0][j] if j < 16 else pair[1][j - 8]

        def pair_body(p, _):
            for b in range(nbuf):
                cl = p * nbuf + b
                wait(cl, b)
                gw = [(gw_v[d, pl.ds(cl * 24, 16)],
                       gw_v[d, pl.ds(cl * 24 + 8, 16)]) for d in range(3)]
                ew = (ew_v[pl.ds(cl * 24, 16)], ew_v[pl.ds(cl * 24 + 8, 16)])
                ns = (ns_v[pl.ds(cl * 24, 16)], ns_v[pl.ds(cl * 24 + 8, 16)])

                orow = lax.rem(cl, chunks_pt) * CH
                for r in range(CH):
                    acc_e = [None] * 4
                    acc_n = [None] * 4
                    for d in range(3):
                        ewd = sc24(ew, r * 3 + d)
                        nsd = sc24(ns, r * 3 + d)
                        for j in range(3):
                            gvw = sc24(gw[d], r * 3 + j)
                            we = gvw * ewd
                            wn = gvw * nsd
                            for v in range(4):
                                rv = rows_v[b, d * 24 + r * 3 + j,
                                            pl.ds(v * 16, 16)]
                                pe = we * rv
                                pn = wn * rv
                                if acc_e[v] is None:
                                    acc_e[v] = pe
                                    acc_n[v] = pn
                                else:
                                    acc_e[v] = acc_e[v] + pe
                                    acc_n[v] = acc_n[v] + pn
                    for v in range(4):
                        out_v[orow + r, pl.ds(v * 16, 16)] = acc_e[v]
                        out_v[orow + r, pl.ds(64 + v * 16, 16)] = acc_n[v]

                @pl.when(cl + nbuf < chunks_pw)
                def _():
                    start(cl + nbuf, b)

                @pl.when(lax.rem(cl, chunks_pt) == chunks_pt - 1)
                def _():
                    t = lax.div(cl, chunks_pt)
                    pltpu.sync_copy(
                        out_v, outh.at[pl.ds(wid * rows_pw + t * OT, OT)])
            return 0

        lax.fori_loop(0, chunks_pw // nbuf, pair_body, 0)

    return functools.partial(
        pl.kernel,
        out_type=jax.ShapeDtypeStruct((NF, 128), f32),
        mesh=mesh,
        scratch_types=[
            pltpu.VMEM((3, seg), i32),
            pltpu.VMEM((3, seg), f32),
            pltpu.VMEM((seg,), f32),
            pltpu.VMEM((seg,), f32),
            pltpu.VMEM((nbuf, 72, 64), f32),
            pltpu.VMEM((OT, 128), f32),
        ] + [pltpu.SemaphoreType.DMA] * nbuf,
        compiler_params=pltpu.CompilerParams(use_tc_tiling_on_sc=False),
    )(body)


def _k1_body(x_ref, g_ref, s_ref):
    i = pl.program_id(0)
    xt = jnp.transpose(x_ref[...].reshape(128, TR))
    rows = lax.broadcasted_iota(i32, (TR, 128), 0) + i * TR
    xt = jnp.where(rows < NV, xt, 0.0)
    g = lax.dot_general(xt, xt, (((0,), (0,)), ((), ())),
                        preferred_element_type=f32)
    s = jnp.sum(xt, axis=0, keepdims=True)
    spad = jnp.concatenate([s, jnp.zeros((7, 128), f32)], axis=0)

    @pl.when(i == 0)
    def _():
        g_ref[...] = g
        s_ref[...] = spad

    @pl.when(i > 0)
    def _():
        g_ref[...] += g
        s_ref[...] += spad


def _k2_body(x_ref, w_ref, b_ref, h_ref):
    i = pl.program_id(0)
    xt = jnp.transpose(x_ref[...].reshape(128, TR))
    rows = lax.broadcasted_iota(i32, (TR, 64), 0) + i * TR
    h = jnp.dot(xt, w_ref[...], preferred_element_type=f32) + b_ref[0:1, :]
    h_ref[...] = jnp.where(rows < NV, jnp.maximum(h, 0.0), 0.0)


def _k6_body(h2_ref, lap_ref, gv_ref, kid_ref, klap_ref, kew_ref, kns_ref,
             y_ref, st_ref):
    i = pl.program_id(0)
    gv = gv_ref[...]
    y = (jnp.dot(h2_ref[...], kid_ref[...], preferred_element_type=f32)
         + jnp.dot(lap_ref[...], klap_ref[...], preferred_element_type=f32)
         + jnp.dot(gv[:, :64], kew_ref[...], preferred_element_type=f32)
         + jnp.dot(gv[:, 64:], kns_ref[...], preferred_element_type=f32))
    y_ref[...] = y
    st = jnp.concatenate([jnp.sum(y, axis=0, keepdims=True),
                          jnp.sum(y * y, axis=0, keepdims=True),
                          jnp.zeros((6, 64), f32)], axis=0)

    @pl.when(i == 0)
    def _():
        st_ref[...] = st

    @pl.when(i > 0)
    def _():
        st_ref[...] += st


def _k7_body(y_ref, s2_ref, t2_ref, w3_ref, b3_ref, z_ref, st_ref):
    i = pl.program_id(0)
    h3 = jnp.maximum(y_ref[...] * s2_ref[0:1, :] + t2_ref[0:1, :], 0.0)
    z = jnp.dot(h3, w3_ref[...], preferred_element_type=f32) + b3_ref[0:1, :]
    rows = lax.broadcasted_iota(i32, (TR, 128), 0) + i * TR
    z = jnp.where(rows < NV, z, 0.0)
    z_ref[...] = z
    st = jnp.concatenate([jnp.sum(z, axis=0, keepdims=True),
                          jnp.sum(z * z, axis=0, keepdims=True),
                          jnp.zeros((6, 128), f32)], axis=0)

    @pl.when(i == 0)
    def _():
        st_ref[...] = st

    @pl.when(i > 0)
    def _():
        st_ref[...] += st


def _k8_body(z_ref, x_ref, s3_ref, t3_ref, o_ref):
    xt = jnp.transpose(x_ref[...].reshape(128, TR))
    o = jnp.maximum(z_ref[...] * s3_ref[0:1, :] + t3_ref[0:1, :] + xt, 0.0)
    o_ref[...] = jnp.transpose(o).reshape(4, 32, TR)


def _row_spec(w):
    return pl.BlockSpec((TR, w), lambda i: (i, 0))


def _full_spec(h, w):
    return pl.BlockSpec((h, w), lambda i: (0, 0))


_GRID = NVP // TR
_GRIDX = -(-NV // TR)


def _tc_call(body, in_specs, out_specs, out_shapes, grid=_GRID):
    return pl.pallas_call(
        body,
        grid=(grid,),
        in_specs=in_specs,
        out_specs=out_specs,
        out_shape=out_shapes,
        compiler_params=pltpu.CompilerParams(
            dimension_semantics=("arbitrary",)),
    )


def _x_spec(clamp=None):
    if clamp is None:
        return pl.BlockSpec((B, IN_CH, TR), lambda i: (0, 0, i))
    return pl.BlockSpec((B, IN_CH, TR),
                        lambda i: (0, 0, jnp.minimum(i, clamp)))


def kernel(x, W1a, b1a, coeffs, W3a, b3a, g1a, be1a, g2a, be2a, g3a, be3a,
           G_rows, G_cols, G_vals, L_rows, L_cols, L_vals,
           F_rows, F_cols, F_vals, EW, NS_):
    N = B * NV
    eyeB = jnp.eye(B, dtype=f32)

    g128, csum8 = _tc_call(
        _k1_body,
        [_x_spec()],
        [_full_spec(128, 128), _full_spec(8, 128)],
        [jax.ShapeDtypeStruct((128, 128), f32),
         jax.ShapeDtypeStruct((8, 128), f32)],
        grid=_GRIDX,
    )(x)
    csum = csum8[0]

    mu_x = csum.reshape(B, IN_CH).sum(0) / N
    Sig = sum(g128[b * IN_CH:(b + 1) * IN_CH, b * IN_CH:(b + 1) * IN_CH]
              for b in range(B)) / N
    mu_h = W1a @ mu_x + b1a
    Eh2 = jnp.einsum('ci,ij,cj->c', W1a, Sig, W1a) + 2 * b1a * (W1a @ mu_x) + b1a ** 2
    s1 = g1a / jnp.sqrt(Eh2 - mu_h ** 2 + EPS)
    W1K = jnp.kron(eyeB, (W1a * s1[:, None]).T)
    b1K = jnp.tile(s1 * (b1a - mu_h) + be1a, B)
    b1K8 = jnp.tile(b1K[None, :], (8, 1))

    (h2,) = _tc_call(
        _k2_body,
        [_x_spec(clamp=_GRIDX - 1), _full_spec(128, 64), _full_spec(8, 64)],
        [_row_spec(64)],
        [jax.ShapeDtypeStruct((NVP, 64), f32)],
    )(x, W1K, b1K8)

    ew_flat = EW.reshape(-1)
    ns_flat = NS_.reshape(-1)

    idxL = jnp.concatenate([L_cols, jnp.asarray(_PAD_L)])
    wL = jnp.concatenate([L_vals, jnp.asarray(_ZW_L)])
    idxF = jnp.concatenate([F_cols, jnp.asarray(_PAD_F)])
    wF = jnp.concatenate([F_vals, jnp.asarray(_ZW_F)])

    lap = _make_sc_gather(NVP, 64, NVP, 7, 1)(h2, idxL, wL)
    gf = _make_sc_gather_g()(h2, G_cols, G_vals, ew_flat, ns_flat)
    gv = _make_sc_gather(NF, 128, NVP, 6, 1)(gf, idxF, wF)

    Ks = [jnp.kron(eyeB, coeffs[j::4, :]) for j in range(4)]
    y, st6 = _tc_call(
        _k6_body,
        [_row_spec(64), _row_spec(64), _row_spec(128),
         _full_spec(64, 64), _full_spec(64, 64), _full_spec(64, 64),
         _full_spec(64, 64)],
        [_row_spec(64), _full_spec(8, 64)],
        [jax.ShapeDtypeStruct((NVP, 64), f32),
         jax.ShapeDtypeStruct((8, 64), f32)],
    )(h2, lap, gv, Ks[0], Ks[1], Ks[2], Ks[3])

    mu_y = st6[0].reshape(B, NECK).sum(0) / N
    var_y = st6[1].reshape(B, NECK).sum(0) / N - mu_y ** 2
    s2 = g2a / jnp.sqrt(var_y + EPS)
    t2 = -mu_y * s2 + be2a
    s2c8 = jnp.tile(jnp.tile(s2, B)[None, :], (8, 1))
    t2c8 = jnp.tile(jnp.tile(t2, B)[None, :], (8, 1))

    W3K = jnp.kron(eyeB, W3a.T)
    b3K8 = jnp.tile(jnp.tile(b3a, B)[None, :], (8, 1))
    z, st7 = _tc_call(
        _k7_body,
        [_row_spec(64), _full_spec(8, 64), _full_spec(8, 64),
         _full_spec(64, 128), _full_spec(8, 128)],
        [_row_spec(128), _full_spec(8, 128)],
        [jax.ShapeDtypeStruct((NVP, 128), f32),
         jax.ShapeDtypeStruct((8, 128), f32)],
    )(y, s2c8, t2c8, W3K, b3K8)

    mu_z = st7[0].reshape(B, OUT_CH).sum(0) / N
    var_z = st7[1].reshape(B, OUT_CH).sum(0) / N - mu_z ** 2
    s3 = g3a / jnp.sqrt(var_z + EPS)
    t3 = -mu_z * s3 + be3a
    s3c8 = jnp.tile(jnp.tile(s3, B)[None, :], (8, 1))
    t3c8 = jnp.tile(jnp.tile(t3, B)[None, :], (8, 1))

    (out,) = _tc_call(
        _k8_body,
        [_row_spec(128), _x_spec(), _full_spec(8, 128), _full_spec(8, 128)],
        [pl.BlockSpec((B, OUT_CH, TR), lambda i: (0, 0, i))],
        [jax.ShapeDtypeStruct((B, OUT_CH, NV), f32)],
        grid=_GRIDX,
    )(z, x, s3c8, t3c8)

    return out
